# baseline (speedup 1.0000x reference)
import functools

import jax
import jax.numpy as jnp
from jax import lax
from jax.experimental import pallas as pl
from jax.experimental.pallas import tpu as pltpu
from jax.experimental.pallas import tpu_sc as plsc

_L = 16
_D = 16
_JB = 8
_IB = 128
_NB = 5


@functools.lru_cache(maxsize=None)
def _build_lookup(n_i: int, n_j: int, n_vocab: int):
  info = plsc.get_sparse_core_info()
  nc, ns = info.num_cores, info.num_subcores
  nw = nc * ns
  assert n_i % (nw * _IB) == 0 and n_j % _JB == 0, (n_i, n_j)
  iblocks_per_w = n_i // (nw * _IB)
  jblocks = n_j // _JB
  n_units = iblocks_per_w * jblocks
  n_groups = _IB // _L
  assert n_units % _NB == 0 and n_units >= 2 * _NB

  mesh = plsc.VectorSubcoreMesh(core_axis_name="c", subcore_axis_name="s")

  @functools.partial(
      pl.kernel,
      mesh=mesh,
      compiler_params=pltpu.CompilerParams(
          needs_layout_passes=False, use_tc_tiling_on_sc=True),
      out_type=jax.ShapeDtypeStruct((n_j, _D, n_i), jnp.float32),
      scratch_types=[
          pltpu.VMEM((_D, n_vocab), jnp.float32),
      ] + [pltpu.VMEM((_JB, _IB), jnp.int32)] * _NB
        + [pltpu.VMEM((_JB, _D, _IB), jnp.float32)] * _NB
        + [pltpu.SemaphoreType.DMA] * (2 * _NB),
  )
  def lookup(table_hbm, idx_hbm, out_hbm, table_v, *bufs):
    idx_v = bufs[:_NB]
    stage_v = bufs[_NB:2 * _NB]
    isem = bufs[2 * _NB:3 * _NB]
    osem = bufs[3 * _NB:4 * _NB]
    wid = lax.axis_index("s") * nc + lax.axis_index("c")
    pltpu.sync_copy(table_hbm, table_v)

    def unit_coords(u):
      ib = u // jblocks
      j0 = pl.multiple_of((u % jblocks) * _JB, _JB)
      i0 = pl.multiple_of((wid * iblocks_per_w + ib) * _IB, _IB)
      return j0, i0

    def idx_copy(u, s):
      j0, i0 = unit_coords(u)
      return pltpu.make_async_copy(
          idx_hbm.at[pl.ds(j0, _JB), pl.ds(i0, _IB)], idx_v[s], isem[s])

    def out_copy(u, s):
      j0, i0 = unit_coords(u)
      return pltpu.make_async_copy(
          stage_v[s], out_hbm.at[pl.ds(j0, _JB), :, pl.ds(i0, _IB)], osem[s])

    def compute(s):
      iv, sv = idx_v[s], stage_v[s]

      @plsc.parallel_loop(0, _JB * n_groups, unroll=2)
      def _(gu):
        jj = lax.shift_right_logical(gu, 3)
        off = pl.multiple_of((gu & (n_groups - 1)) * _L, _L)
        src = iv[jj, pl.ds(off, _L)]
        for d in range(_D):
          row = jnp.full((_L,), d, jnp.int32)
          sv[jj, d, pl.ds(off, _L)] = plsc.load_gather(table_v, [row, src])

    for s in range(_NB):
      idx_copy(s, s).start()
    for s in range(_NB):
      idx_copy(s, s).wait()
      compute(s)
      out_copy(s, s).start()
      idx_copy(s + _NB, s).start()

    last = n_units - 1

    def ring_body(p, carry):
      for s in range(_NB):
        u = _NB * p + s
        idx_copy(u, s).wait()
        out_copy(u, s).wait()
        compute(s)
        out_copy(u, s).start()
        up = jnp.minimum(u + _NB, last)
        idx_copy(up, s).start()
      return carry

    lax.fori_loop(1, n_units // _NB, ring_body, 0)

    for s in range(_NB):
      idx_copy(last, s).wait()
      out_copy(last, s).wait()

  return lookup


def kernel(x, embedding):
  n_i, n_j = x.shape
  xt = x.T.astype(jnp.int32)
  emb = embedding.astype(jnp.float32).T
  fn = _build_lookup(n_i, n_j, emb.shape[1])
  out = fn(emb, xt)
  return out.transpose(2, 0, 1)


# speedup vs baseline: 1.0087x; 1.0087x over previous
import functools

import jax
import jax.numpy as jnp
from jax import lax
from jax.experimental import pallas as pl
from jax.experimental.pallas import tpu as pltpu
from jax.experimental.pallas import tpu_sc as plsc

_L = 16
_D = 16
_JB = 8
_IB = 128
_NB = 4


@functools.lru_cache(maxsize=None)
def _build_lookup(n_i: int, n_j: int, n_vocab: int):
  info = plsc.get_sparse_core_info()
  nc, ns = info.num_cores, info.num_subcores
  nw = nc * ns
  assert n_i % (nw * _IB) == 0 and n_j % _JB == 0, (n_i, n_j)
  iblocks_per_w = n_i // (nw * _IB)
  jblocks = n_j // _JB
  n_units = iblocks_per_w * jblocks
  n_groups = _IB // _L
  assert n_units % _NB == 0 and n_units >= 2 * _NB

  mesh = plsc.VectorSubcoreMesh(core_axis_name="c", subcore_axis_name="s")

  @functools.partial(
      pl.kernel,
      mesh=mesh,
      compiler_params=pltpu.CompilerParams(
          needs_layout_passes=False, use_tc_tiling_on_sc=True),
      out_type=jax.ShapeDtypeStruct((n_j, _D, n_i), jnp.float32),
      scratch_types=[
          pltpu.VMEM((_D, n_vocab), jnp.float32),
      ] + [pltpu.VMEM((_JB, _IB), jnp.int32)] * _NB
        + [pltpu.VMEM((_JB, _D, _IB), jnp.float32)] * _NB
        + [pltpu.SemaphoreType.DMA] * (2 * _NB),
  )
  def lookup(table_hbm, idx_hbm, out_hbm, table_v, *bufs):
    idx_v = bufs[:_NB]
    stage_v = bufs[_NB:2 * _NB]
    isem = bufs[2 * _NB:3 * _NB]
    osem = bufs[3 * _NB:4 * _NB]
    wid = lax.axis_index("s") * nc + lax.axis_index("c")
    pltpu.sync_copy(table_hbm, table_v)

    def unit_coords(u):
      ib = u // jblocks
      j0 = pl.multiple_of((u % jblocks) * _JB, _JB)
      i0 = pl.multiple_of((wid * iblocks_per_w + ib) * _IB, _IB)
      return j0, i0

    def idx_copy(u, s):
      j0, i0 = unit_coords(u)
      return pltpu.make_async_copy(
          idx_hbm.at[pl.ds(j0, _JB), pl.ds(i0, _IB)], idx_v[s], isem[s])

    def out_copy(u, s):
      j0, i0 = unit_coords(u)
      return pltpu.make_async_copy(
          stage_v[s], out_hbm.at[pl.ds(j0, _JB), :, pl.ds(i0, _IB)], osem[s])

    def compute(s):
      iv, sv = idx_v[s], stage_v[s]

      @plsc.parallel_loop(0, _JB * n_groups, unroll=2)
      def _(gu):
        jj = lax.shift_right_logical(gu, 3)
        off = pl.multiple_of((gu & (n_groups - 1)) * _L, _L)
        src = iv[jj, pl.ds(off, _L)]
        for d in range(_D):
          row = jnp.full((_L,), d, jnp.int32)
          sv[jj, d, pl.ds(off, _L)] = plsc.load_gather(table_v, [row, src])

    for s in range(_NB):
      idx_copy(s, s).start()
    for s in range(_NB):
      idx_copy(s, s).wait()
      compute(s)
      out_copy(s, s).start()
      idx_copy(s + _NB, s).start()

    last = n_units - 1

    def ring_body(p, carry):
      for s in range(_NB):
        u = _NB * p + s
        idx_copy(u, s).wait()
        out_copy(u, s).wait()
        compute(s)
        out_copy(u, s).start()
        up = jnp.minimum(u + _NB, last)
        idx_copy(up, s).start()
      return carry

    lax.fori_loop(1, n_units // _NB, ring_body, 0)

    for s in range(_NB):
      idx_copy(last, s).wait()
      out_copy(last, s).wait()

  return lookup


def kernel(x, embedding):
  n_i, n_j = x.shape
  xt = x.T.astype(jnp.int32)
  emb = embedding.astype(jnp.float32).T
  fn = _build_lookup(n_i, n_j, emb.shape[1])
  out = fn(emb, xt)
  return out.transpose(2, 0, 1)
